# baseline (speedup 1.0000x reference)
import jax
import jax.numpy as jnp
from jax import lax
from jax.experimental import pallas as pl

B = 8
NL = 13
L = 512
D = 768
NBINS = 320
DH = D // 2


def _gather_counts(e):
    ones_col = jnp.ones((L, 1), jnp.bfloat16)
    dn_nn = (((1,), (0,)), ((), ()))
    c = lax.dot_general(e, ones_col, dn_nn,
                        preferred_element_type=jnp.float32)
    hi = c.astype(jnp.bfloat16)
    lo = (c - hi.astype(jnp.float32)).astype(jnp.bfloat16)
    hl = jnp.concatenate([hi, lo], axis=1)
    dn_cc = (((0,), (0,)), ((), ()))
    fhl = lax.dot_general(hl, e, dn_cc,
                          preferred_element_type=jnp.float32)
    return fhl[0:1, :] + fhl[1:2, :]


def _body(vq_ref, xlo_ref, xhi_ref, o_ref):
    v = vq_ref[0]
    iota_s = lax.broadcasted_iota(jnp.int32, (NBINS, L), 0)
    ex = (v[0:1, :] == iota_s).astype(jnp.bfloat16)
    ey = (v[1:2, :] == iota_s).astype(jnp.bfloat16)
    f = _gather_counts(ex) + _gather_counts(ey)
    r = 1.0 / f
    w = jnp.transpose(r * (1.0 / jnp.sum(r)))
    olo = jnp.sum(xlo_ref[0, 0] * w, axis=0, keepdims=True)
    ohi = jnp.sum(xhi_ref[0, 0] * w, axis=0, keepdims=True)
    o_ref[0] = jnp.concatenate([olo, ohi], axis=1)


def kernel(input_feature, input_lengths, vq_indices):
    del input_lengths
    vq = jnp.transpose(vq_indices.astype(jnp.int32), (0, 2, 1))
    out = pl.pallas_call(
        _body,
        grid=(B,),
        in_specs=[
            pl.BlockSpec((1, 2, L), lambda b: (b, 0, 0)),
            pl.BlockSpec((1, 1, L, DH), lambda b: (b, NL - 1, 0, 0)),
            pl.BlockSpec((1, 1, L, DH), lambda b: (b, NL - 1, 0, 1)),
        ],
        out_specs=pl.BlockSpec((1, 1, D), lambda b: (b, 0, 0)),
        out_shape=jax.ShapeDtypeStruct((B, 1, D), jnp.float32),
    )(vq, input_feature, input_feature)
    return out.reshape(B, D)


# speedup vs baseline: 1.3174x; 1.3174x over previous
import jax
import jax.numpy as jnp
from jax import lax
from jax.experimental import pallas as pl

B = 8
NL = 13
L = 512
D = 768
NBINS = 320
DH = D // 2
SPB = 2


def _weights(v):
    iota_s = lax.broadcasted_iota(jnp.int32, (2 * NBINS, L), 0)
    is_x = iota_s < NBINS
    iota_mod = jnp.where(is_x, iota_s, iota_s - NBINS)
    tgt = jnp.where(is_x, v[0:1, :], v[1:2, :])
    et = (tgt == iota_mod).astype(jnp.bfloat16)
    ones_col = jnp.ones((L, 1), jnp.bfloat16)
    dn_nn = (((1,), (0,)), ((), ()))
    c = lax.dot_general(et, ones_col, dn_nn,
                        preferred_element_type=jnp.float32)
    hi = c.astype(jnp.bfloat16)
    lo = (c - hi.astype(jnp.float32)).astype(jnp.bfloat16)
    hl = jnp.concatenate([hi, lo], axis=1)
    dn_cc = (((0,), (0,)), ((), ()))
    fhl = lax.dot_general(hl, et, dn_cc,
                          preferred_element_type=jnp.float32)
    f = fhl[0:1, :] + fhl[1:2, :]
    r = 1.0 / f
    return jnp.transpose(r * (1.0 / jnp.sum(r)))


def _body(vq_ref, xlo_ref, xhi_ref, o_ref):
    for i in range(SPB):
        w = _weights(vq_ref[i])
        olo = jnp.sum(xlo_ref[i, 0] * w, axis=0, keepdims=True)
        ohi = jnp.sum(xhi_ref[i, 0] * w, axis=0, keepdims=True)
        o_ref[i] = jnp.concatenate([olo, ohi], axis=1)


def kernel(input_feature, input_lengths, vq_indices):
    del input_lengths
    vq = jnp.transpose(vq_indices.astype(jnp.int32), (0, 2, 1))
    out = pl.pallas_call(
        _body,
        grid=(B // SPB,),
        in_specs=[
            pl.BlockSpec((SPB, 2, L), lambda b: (b, 0, 0)),
            pl.BlockSpec((SPB, 1, L, DH), lambda b: (b, NL - 1, 0, 0)),
            pl.BlockSpec((SPB, 1, L, DH), lambda b: (b, NL - 1, 0, 1)),
        ],
        out_specs=pl.BlockSpec((SPB, 1, D), lambda b: (b, 0, 0)),
        out_shape=jax.ShapeDtypeStruct((B, 1, D), jnp.float32),
    )(vq, input_feature, input_feature)
    return out.reshape(B, D)


# speedup vs baseline: 1.3224x; 1.0038x over previous
import jax
import jax.numpy as jnp
from jax import lax
from jax.experimental import pallas as pl

B = 8
NL = 13
L = 512
D = 768
NBINS = 320
DH = D // 2
SPB = 4


def _weights(v):
    iota_s = lax.broadcasted_iota(jnp.int32, (2 * NBINS, L), 0)
    is_x = iota_s < NBINS
    iota_mod = jnp.where(is_x, iota_s, iota_s - NBINS)
    tgt = jnp.where(is_x, v[0:1, :], v[1:2, :])
    et = (tgt == iota_mod).astype(jnp.bfloat16)
    ones_col = jnp.ones((L, 1), jnp.bfloat16)
    dn_nn = (((1,), (0,)), ((), ()))
    c = lax.dot_general(et, ones_col, dn_nn,
                        preferred_element_type=jnp.float32)
    hi = c.astype(jnp.bfloat16)
    lo = (c - hi.astype(jnp.float32)).astype(jnp.bfloat16)
    hl = jnp.concatenate([hi, lo], axis=1)
    dn_cc = (((0,), (0,)), ((), ()))
    fhl = lax.dot_general(hl, et, dn_cc,
                          preferred_element_type=jnp.float32)
    f = fhl[0:1, :] + fhl[1:2, :]
    r = 1.0 / f
    return jnp.transpose(r * (1.0 / jnp.sum(r)))


def _body(vq_ref, xlo_ref, xhi_ref, o_ref):
    for i in range(SPB):
        w = _weights(vq_ref[i])
        olo = jnp.sum(xlo_ref[i, 0] * w, axis=0, keepdims=True)
        ohi = jnp.sum(xhi_ref[i, 0] * w, axis=0, keepdims=True)
        o_ref[i] = jnp.concatenate([olo, ohi], axis=1)


def kernel(input_feature, input_lengths, vq_indices):
    del input_lengths
    vq = jnp.transpose(vq_indices.astype(jnp.int32), (0, 2, 1))
    out = pl.pallas_call(
        _body,
        grid=(B // SPB,),
        in_specs=[
            pl.BlockSpec((SPB, 2, L), lambda b: (b, 0, 0)),
            pl.BlockSpec((SPB, 1, L, DH), lambda b: (b, NL - 1, 0, 0)),
            pl.BlockSpec((SPB, 1, L, DH), lambda b: (b, NL - 1, 0, 1)),
        ],
        out_specs=pl.BlockSpec((SPB, 1, D), lambda b: (b, 0, 0)),
        out_shape=jax.ShapeDtypeStruct((B, 1, D), jnp.float32),
    )(vq, input_feature, input_feature)
    return out.reshape(B, D)


# speedup vs baseline: 1.5604x; 1.1799x over previous
import jax
import jax.numpy as jnp
from jax import lax
from jax.experimental import pallas as pl

B = 8
NL = 13
L = 512
D = 768
NBINS = 320
DH = D // 2
SPB = 2


def _weights(v):
    iota_s = lax.broadcasted_iota(jnp.int32, (2 * NBINS, L), 0)
    is_x = iota_s < NBINS
    iota_mod = jnp.where(is_x, iota_s, iota_s - NBINS)
    tgt = jnp.where(is_x, v[0:1, :], v[1:2, :])
    et = (tgt == iota_mod).astype(jnp.bfloat16)
    ones_col = jnp.ones((L, 1), jnp.bfloat16)
    dn_nn = (((1,), (0,)), ((), ()))
    c = lax.dot_general(et, ones_col, dn_nn,
                        preferred_element_type=jnp.float32)
    cs = (c - 256.0).astype(jnp.bfloat16)
    dn_cc = (((0,), (0,)), ((), ()))
    f = lax.dot_general(cs, et, dn_cc,
                        preferred_element_type=jnp.float32)
    r = 1.0 / (f + 512.0)
    return jnp.transpose(r * (1.0 / jnp.sum(r)))


def _body(vq_ref, xlo_ref, xhi_ref, o_ref):
    for i in range(SPB):
        w = _weights(vq_ref[i])
        olo = jnp.sum(xlo_ref[i, 0] * w, axis=0, keepdims=True)
        ohi = jnp.sum(xhi_ref[i, 0] * w, axis=0, keepdims=True)
        o_ref[i] = jnp.concatenate([olo, ohi], axis=1)


def kernel(input_feature, input_lengths, vq_indices):
    del input_lengths
    vq = jnp.transpose(vq_indices.astype(jnp.int32), (0, 2, 1))
    out = pl.pallas_call(
        _body,
        grid=(B // SPB,),
        in_specs=[
            pl.BlockSpec((SPB, 2, L), lambda b: (b, 0, 0)),
            pl.BlockSpec((SPB, 1, L, DH), lambda b: (b, NL - 1, 0, 0)),
            pl.BlockSpec((SPB, 1, L, DH), lambda b: (b, NL - 1, 0, 1)),
        ],
        out_specs=pl.BlockSpec((SPB, 1, D), lambda b: (b, 0, 0)),
        out_shape=jax.ShapeDtypeStruct((B, 1, D), jnp.float32),
    )(vq, input_feature, input_feature)
    return out.reshape(B, D)


# speedup vs baseline: 1.5746x; 1.0091x over previous
import jax
import jax.numpy as jnp
from jax import lax
from jax.experimental import pallas as pl

B = 8
NL = 13
L = 512
D = 768
NBINS = 320
DH = D // 2
SPB = 2
L2 = SPB * L


def _body(vq_ref, xlo_ref, xhi_ref, o_ref):
    vx = jnp.concatenate([vq_ref[0, 0:1, :], vq_ref[1, 0:1, :]], axis=1)
    vy = jnp.concatenate([vq_ref[0, 1:2, :], vq_ref[1, 1:2, :]], axis=1)
    iota_s = lax.broadcasted_iota(jnp.int32, (2 * NBINS, L2), 0)
    is_x = iota_s < NBINS
    iota_mod = jnp.where(is_x, iota_s, iota_s - NBINS)
    tgt = jnp.where(is_x, vx, vy)
    et = (tgt == iota_mod).astype(jnp.bfloat16)
    lane = lax.broadcasted_iota(jnp.int32, (L2, SPB), 0)
    col = lax.broadcasted_iota(jnp.int32, (L2, SPB), 1)
    bd1 = ((lane // L) == col).astype(jnp.bfloat16)
    dn_nn = (((1,), (0,)), ((), ()))
    c = lax.dot_general(et, bd1, dn_nn,
                        preferred_element_type=jnp.float32)
    cs = (c - 256.0).astype(jnp.bfloat16)
    dn_cc = (((0,), (0,)), ((), ()))
    g = lax.dot_general(cs, et, dn_cc,
                        preferred_element_type=jnp.float32)
    for i in range(SPB):
        f = g[i:i + 1, i * L:(i + 1) * L] + 512.0
        r = 1.0 / f
        w = jnp.transpose(r * (1.0 / jnp.sum(r)))
        olo = jnp.sum(xlo_ref[i, 0] * w, axis=0, keepdims=True)
        ohi = jnp.sum(xhi_ref[i, 0] * w, axis=0, keepdims=True)
        o_ref[i] = jnp.concatenate([olo, ohi], axis=1)


def kernel(input_feature, input_lengths, vq_indices):
    del input_lengths
    vq = jnp.transpose(vq_indices.astype(jnp.int32), (0, 2, 1))
    out = pl.pallas_call(
        _body,
        grid=(B // SPB,),
        in_specs=[
            pl.BlockSpec((SPB, 2, L), lambda b: (b, 0, 0)),
            pl.BlockSpec((SPB, 1, L, DH), lambda b: (b, NL - 1, 0, 0)),
            pl.BlockSpec((SPB, 1, L, DH), lambda b: (b, NL - 1, 0, 1)),
        ],
        out_specs=pl.BlockSpec((SPB, 1, D), lambda b: (b, 0, 0)),
        out_shape=jax.ShapeDtypeStruct((B, 1, D), jnp.float32),
    )(vq, input_feature, input_feature)
    return out.reshape(B, D)
